# baseline (speedup 1.0000x reference)
import functools

import jax
import jax.numpy as jnp
from jax import lax
from jax.experimental import pallas as pl
from jax.experimental.pallas import tpu as pltpu
from jax.experimental.pallas import tpu_sc as plsc

VOCAB = 1000000
DIM = 64
B = 4096
L = 50

N = B * L
NC, NS = 2, 16
NW = NC * NS
PER_W = N // NW
VBT = 8192
RANGE = VBT // NW
CAP = 212992
NSLAB = NW * CAP
CHUNK = 2048
DUMP = N
PADSTG = RANGE * 16

_params = pltpu.CompilerParams(use_tc_tiling_on_sc=True,
                               needs_layout_passes=False)
_mesh = functools.partial(plsc.VectorSubcoreMesh,
                          core_axis_name="c", subcore_axis_name="s")


def _wid():
    return lax.axis_index("s") * NC + lax.axis_index("c")


def _iota():
    return lax.iota(jnp.int32, 16)


def _k1_hist(idx_hbm, hists_hbm, idx_v, hist_v, sem):
    w = _wid()
    iota16 = _iota()
    ones16 = jnp.ones((16,), jnp.int32)
    masks = [iota16 == j for j in range(16)]
    pltpu.sync_copy(idx_hbm.at[pl.ds(w * PER_W, PER_W)], idx_v)
    zero16 = jnp.zeros((16,), jnp.int32)
    for g in range(VBT // 16):
        hist_v[pl.ds(g * 16, 16)] = zero16

    def body(g, _):
        vb16 = idx_v[pl.ds(g * 16, 16)] >> 7
        for j in range(16):
            plsc.addupdate_scatter(hist_v, [vb16], ones16, mask=masks[j])
        return 0

    lax.fori_loop(0, PER_W // 16, body, 0)
    pltpu.sync_copy(hist_v, hists_hbm.at[w])


def _k2_offs(hists_hbm, offs_hbm, cum_hbm, cnts_hbm,
             hblk_v, offs_v, cnt_v, cum_v, sem):
    w = _wid()
    iota16 = _iota()
    for t in range(NW):
        pltpu.sync_copy(hists_hbm.at[t, pl.ds(w * RANGE, RANGE)],
                        hblk_v.at[pl.ds(t * RANGE, RANGE)])
    for g in range(RANGE // 16):
        sl = pl.ds(g * 16, 16)
        acc = jnp.zeros((16,), jnp.int32)
        for t in range(NW):
            offs_v[pl.ds(t * RANGE + g * 16, 16)] = acc
            acc = acc + hblk_v[pl.ds(t * RANGE + g * 16, 16)]
        cnt_v[sl] = acc

    carry = jnp.int32(0)
    for g in range(RANGE // 16):
        sl = pl.ds(g * 16, 16)
        cntp = (cnt_v[sl] + 15) & ~15
        pre = plsc.cumsum(cntp)
        cum_v[sl] = carry + pre - cntp
        carry = carry + pre[15]
    plsc.store_scatter(cum_v, [jnp.full((16,), RANGE, jnp.int32)],
                       jnp.full((16,), 1, jnp.int32) * carry,
                       mask=iota16 == 0)

    base = w * CAP
    for g in range(RANGE // 16):
        cb = cum_v[pl.ds(g * 16, 16)] + base
        for t in range(NW):
            sl = pl.ds(t * RANGE + g * 16, 16)
            offs_v[sl] = offs_v[sl] + cb
    for t in range(NW):
        pltpu.sync_copy(offs_v.at[pl.ds(t * RANGE, RANGE)],
                        offs_hbm.at[t, pl.ds(w * RANGE, RANGE)])
    pltpu.sync_copy(cum_v, cum_hbm.at[w])
    pltpu.sync_copy(cnt_v, cnts_hbm.at[w])


def _k3_route(idx_hbm, offs_hbm, cum_hbm, cnts_hbm, vals_hbm, poss_hbm,
              idx_v, offr_v, dest_v, posb_v, cum_v, cnt_v,
              pval_v, ppos_v, pslot_v, sem):
    w = _wid()
    iota16 = _iota()
    ones16 = jnp.ones((16,), jnp.int32)
    masks = [iota16 == j for j in range(16)]
    pltpu.sync_copy(idx_hbm.at[pl.ds(w * PER_W, PER_W)], idx_v)
    pltpu.sync_copy(offs_hbm.at[w], offr_v)

    def alloc(g, _):
        sl = pl.ds(g * 16, 16)
        vb16 = idx_v[sl] >> 7
        dest16 = jnp.zeros((16,), jnp.int32)
        for j in range(16):
            o16 = plsc.load_gather(offr_v, [vb16], mask=masks[j])
            dest16 = jnp.where(masks[j], o16, dest16)
            plsc.addupdate_scatter(offr_v, [vb16], ones16, mask=masks[j])
        dest_v[sl] = dest16
        posb_v[sl] = iota16 + (w * PER_W + g * 16)
        return 0

    lax.fori_loop(0, PER_W // 16, alloc, 0)

    pltpu.sync_copy(cum_hbm.at[w], cum_v)
    pltpu.sync_copy(cnts_hbm.at[w], cnt_v)
    trash = w * CAP + CAP - 16
    dump16 = jnp.full((16,), DUMP, jnp.int32)
    for g in range(RANGE // 16):
        sl = pl.ds(g * 16, 16)
        cnt16 = cnt_v[sl]
        pc16 = ((cnt16 + 15) & ~15) - cnt16
        sb16 = (w * CAP) + cum_v[sl] + cnt16
        vid16 = (w * RANGE + g * 16 + iota16) << 7
        for j in range(16):
            slots = jnp.where(iota16 < pc16[j], sb16[j] + iota16,
                              trash + iota16)
            st = pl.ds((g * 16 + j) * 16, 16)
            pslot_v[st] = slots
            pval_v[st] = jnp.full((16,), 1, jnp.int32) * vid16[j]
            ppos_v[st] = dump16

    pltpu.async_copy(idx_v, vals_hbm.at[dest_v], sem).wait()
    pltpu.async_copy(posb_v, poss_hbm.at[dest_v], sem).wait()
    pltpu.async_copy(pval_v, vals_hbm.at[pslot_v], sem).wait()
    pltpu.async_copy(ppos_v, poss_hbm.at[pslot_v], sem).wait()


def _k4_extract(embT_hbm, tail_hbm, vals_hbm, poss_hbm, cum_hbm, out_hbm,
                valc_v, posc_v, col_v, stage_v, pstage_v, cum_v, sem, osem):
    w = _wid()
    iota16 = _iota()
    pltpu.sync_copy(cum_hbm.at[w], cum_v)
    total = cum_v[pl.ds(RANGE, 16)][0]
    base = w * CAP
    dump16 = jnp.full((16,), DUMP, jnp.int32)
    for r in range(8):
        pstage_v[pl.ds(r * 16, 16)] = dump16

    def chunk_body(c, cached):
        coff = base + c * CHUNK
        pltpu.sync_copy(vals_hbm.at[pl.ds(coff, CHUNK)], valc_v)
        pltpu.sync_copy(poss_hbm.at[pl.ds(coff, CHUNK)], posc_v)
        ngrp = jnp.minimum((total - c * CHUNK) >> 4, CHUNK >> 4)

        def grp_body(g, cached2):
            sl = pl.ds(g * 16, 16)
            val16 = valc_v[sl]
            vbj = val16[0] >> 7
            cv16 = val16 & 127

            def load_col():
                def full_block():
                    pltpu.sync_copy(
                        embT_hbm.at[:, pl.ds(vbj * 128, 128)], col_v)

                def tail_block():
                    pltpu.sync_copy(tail_hbm, col_v)

                pl.when(vbj < 7812)(full_block)
                pl.when(vbj == 7812)(tail_block)

            pl.when(vbj != cached2)(load_col)
            row16 = iota16 + (g & 7) * 16
            for d in range(DIM):
                v = plsc.load_gather(col_v,
                                     [jnp.full((16,), d, jnp.int32), cv16])
                plsc.store_scatter(stage_v,
                                   [row16, jnp.full((16,), d, jnp.int32)], v)
            pstage_v[pl.ds((g & 7) * 16, 16)] = posc_v[sl]

            def flush():
                pltpu.async_copy(stage_v, out_hbm.at[pstage_v], osem).wait()

            pl.when((g & 7) == 7)(flush)
            return vbj

        return lax.fori_loop(0, ngrp, grp_body, cached)

    nchunks = (total + CHUNK - 1) >> 11
    lax.fori_loop(0, nchunks, chunk_body, jnp.int32(-1))
    pltpu.async_copy(stage_v, out_hbm.at[pstage_v], osem).wait()


@jax.jit
def _embed_lookup(embT, tail, idx):
    k1 = pl.kernel(
        _k1_hist, mesh=_mesh(), compiler_params=_params,
        out_type=jax.ShapeDtypeStruct((NW, VBT), jnp.int32),
        scratch_types=[pltpu.VMEM((PER_W,), jnp.int32),
                       pltpu.VMEM((VBT,), jnp.int32),
                       pltpu.SemaphoreType.DMA])
    hists = k1(idx)

    k2 = pl.kernel(
        _k2_offs, mesh=_mesh(), compiler_params=_params,
        out_type=(jax.ShapeDtypeStruct((NW, VBT), jnp.int32),
                  jax.ShapeDtypeStruct((NW, RANGE + 128), jnp.int32),
                  jax.ShapeDtypeStruct((NW, RANGE), jnp.int32)),
        scratch_types=[pltpu.VMEM((NW * RANGE,), jnp.int32),
                       pltpu.VMEM((NW * RANGE,), jnp.int32),
                       pltpu.VMEM((RANGE,), jnp.int32),
                       pltpu.VMEM((RANGE + 128,), jnp.int32),
                       pltpu.SemaphoreType.DMA])
    offs, cum, cnts = k2(hists)

    k3 = pl.kernel(
        _k3_route, mesh=_mesh(), compiler_params=_params,
        out_type=(jax.ShapeDtypeStruct((NSLAB,), jnp.int32),
                  jax.ShapeDtypeStruct((NSLAB,), jnp.int32)),
        scratch_types=[pltpu.VMEM((PER_W,), jnp.int32),
                       pltpu.VMEM((VBT,), jnp.int32),
                       pltpu.VMEM((PER_W,), jnp.int32),
                       pltpu.VMEM((PER_W,), jnp.int32),
                       pltpu.VMEM((RANGE + 128,), jnp.int32),
                       pltpu.VMEM((RANGE,), jnp.int32),
                       pltpu.VMEM((PADSTG,), jnp.int32),
                       pltpu.VMEM((PADSTG,), jnp.int32),
                       pltpu.VMEM((PADSTG,), jnp.int32),
                       pltpu.SemaphoreType.DMA])
    vals, poss = k3(idx, offs, cum, cnts)

    k4 = pl.kernel(
        _k4_extract, mesh=_mesh(), compiler_params=_params,
        out_type=jax.ShapeDtypeStruct((N + 16, 2 * DIM), jnp.float32),
        scratch_types=[pltpu.VMEM((CHUNK,), jnp.int32),
                       pltpu.VMEM((CHUNK,), jnp.int32),
                       pltpu.VMEM((DIM, 128), jnp.float32),
                       pltpu.VMEM((128, 2 * DIM), jnp.float32),
                       pltpu.VMEM((128,), jnp.int32),
                       pltpu.VMEM((RANGE + 128,), jnp.int32),
                       pltpu.SemaphoreType.DMA,
                       pltpu.SemaphoreType.DMA])
    outp = k4(embT, tail, vals, poss, cum)
    return outp[:N, :DIM]


def kernel(emb, inputs):
    tail = jnp.pad(emb[VOCAB - 64:], ((0, 64), (0, 0))).T
    out = _embed_lookup(emb.T, tail, inputs.reshape(N))
    return out.reshape(B, L, DIM)


# speedup vs baseline: 6.2737x; 6.2737x over previous
import functools

import jax
import jax.numpy as jnp
from jax import lax
from jax.experimental import pallas as pl
from jax.experimental.pallas import tpu as pltpu
from jax.experimental.pallas import tpu_sc as plsc

VOCAB = 1000000
DIM = 64
B = 4096
L = 50

N = B * L
NC, NS = 2, 16
NW = NC * NS
PER_W = N // NW
CHUNK = 800
NCHUNK = PER_W // CHUNK


def _gather_kernel(emb_hbm, idx_hbm, out_hbm,
                   idx_v0, idx_v1, rows_v0, rows_v1, sem0, sem1):
    wid = lax.axis_index("s") * NC + lax.axis_index("c")
    base = wid * PER_W
    idx_bufs = (idx_v0, idx_v1)
    rows_bufs = (rows_v0, rows_v1)
    sems = (sem0, sem1)

    pltpu.sync_copy(idx_hbm.at[pl.ds(base, CHUNK)], idx_v0)
    pltpu.async_copy(emb_hbm.at[idx_v0], rows_v0, sem0)

    for j in range(NCHUNK):
        cur = j % 2
        nxt = (j + 1) % 2
        if j + 1 < NCHUNK:
            pltpu.sync_copy(
                idx_hbm.at[pl.ds(base + (j + 1) * CHUNK, CHUNK)],
                idx_bufs[nxt])
            pltpu.async_copy(emb_hbm.at[idx_bufs[nxt]], rows_bufs[nxt],
                             sems[nxt])
        pltpu.make_async_copy(emb_hbm.at[idx_bufs[cur]], rows_bufs[cur],
                              sems[cur]).wait()
        pltpu.sync_copy(rows_bufs[cur],
                        out_hbm.at[pl.ds(base + j * CHUNK, CHUNK)])


@jax.jit
def _embed_lookup(emb, idx_flat):
    mesh = plsc.VectorSubcoreMesh(core_axis_name="c", subcore_axis_name="s")
    k = pl.kernel(
        _gather_kernel,
        mesh=mesh,
        compiler_params=pltpu.CompilerParams(use_tc_tiling_on_sc=False),
        out_type=jax.ShapeDtypeStruct((N, DIM), jnp.float32),
        scratch_types=[
            pltpu.VMEM((CHUNK,), jnp.int32),
            pltpu.VMEM((CHUNK,), jnp.int32),
            pltpu.VMEM((CHUNK, DIM), jnp.float32),
            pltpu.VMEM((CHUNK, DIM), jnp.float32),
            pltpu.SemaphoreType.DMA,
            pltpu.SemaphoreType.DMA,
        ],
    )
    return k(emb, idx_flat)


def kernel(emb, inputs):
    out = _embed_lookup(emb, inputs.reshape(N))
    return out.reshape(B, L, DIM)
